# initial kernel scaffold (unmeasured)
import jax
import jax.numpy as jnp
from jax import lax
from jax.experimental import pallas as pl
from jax.experimental.pallas import tpu as pltpu

N_DEV = 8
M = 768
N = 768
CHUNK = M // N_DEV

_RS_OFFS = (0, 384, 576)


def kernel(A, B):
    def body(a_ref, b_ref, out_ref, recv_ref, send_sems, recv_sems):
        my = lax.axis_index("i")

        out_ref[:, :] = jnp.dot(
            a_ref[:, :], b_ref[:, :], preferred_element_type=jnp.float32
        )

        base = jnp.int32(0)
        for k in range(3):
            half = M >> (k + 1)
            bit = (my >> (2 - k)) & 1
            partner = my ^ (4 >> k)
            keep_off = base + bit * half
            send_off = base + (1 - bit) * half
            rdma = pltpu.make_async_remote_copy(
                src_ref=out_ref.at[pl.ds(send_off, half), :],
                dst_ref=recv_ref.at[pl.ds(_RS_OFFS[k], half), :],
                send_sem=send_sems.at[k],
                recv_sem=recv_sems.at[k],
                device_id=(partner,),
                device_id_type=pl.DeviceIdType.MESH,
            )
            rdma.start()
            rdma.wait()
            out_ref[pl.ds(keep_off, half), :] = (
                out_ref[pl.ds(keep_off, half), :]
                + recv_ref[pl.ds(_RS_OFFS[k], half), :]
            )
            base = keep_off

        for k in range(3):
            cur_len = CHUNK << k
            base_rows = ((my >> k) << k) * CHUNK
            partner = my ^ (1 << k)
            rdma = pltpu.make_async_remote_copy(
                src_ref=out_ref.at[pl.ds(base_rows, cur_len), :],
                dst_ref=out_ref.at[pl.ds(base_rows, cur_len), :],
                send_sem=send_sems.at[3 + k],
                recv_sem=recv_sems.at[3 + k],
                device_id=(partner,),
                device_id_type=pl.DeviceIdType.MESH,
            )
            rdma.start()
            rdma.wait()

    return pl.pallas_call(
        body,
        out_shape=jax.ShapeDtypeStruct((M, N), jnp.float32),
        in_specs=[
            pl.BlockSpec(memory_space=pltpu.VMEM),
            pl.BlockSpec(memory_space=pltpu.VMEM),
        ],
        out_specs=pl.BlockSpec(memory_space=pltpu.VMEM),
        scratch_shapes=[
            pltpu.VMEM((672, N), jnp.float32),
            pltpu.SemaphoreType.DMA((6,)),
            pltpu.SemaphoreType.DMA((6,)),
        ],
        compiler_params=pltpu.CompilerParams(collective_id=0),
    )(A, B)


# baseline (device time: 66767 ns/iter reference)
import jax
import jax.numpy as jnp
from jax import lax
from jax.experimental import pallas as pl
from jax.experimental.pallas import tpu as pltpu

N_DEV = 8
M = 768
N = 768
CHUNK = M // N_DEV

_RS_OFFS = (0, 384, 576)


def kernel(A, B):
    def body(a_ref, b_ref, out_ref, recv_ref, send_sems, recv_sems):
        my = lax.axis_index("i")

        out_ref[:, :] = jnp.dot(
            a_ref[:, :], b_ref[:, :], preferred_element_type=jnp.float32
        )

        base = jnp.int32(0)
        for k in range(3):
            half = M >> (k + 1)
            bit = (my >> (2 - k)) & 1
            partner = my ^ (4 >> k)
            keep_off = base + bit * half
            send_off = base + (1 - bit) * half
            rdma = pltpu.make_async_remote_copy(
                src_ref=out_ref.at[pl.ds(send_off, half), :],
                dst_ref=recv_ref.at[pl.ds(_RS_OFFS[k], half), :],
                send_sem=send_sems.at[k],
                recv_sem=recv_sems.at[k],
                device_id=(partner,),
                device_id_type=pl.DeviceIdType.MESH,
            )
            rdma.start()
            rdma.wait()
            out_ref[pl.ds(keep_off, half), :] = (
                out_ref[pl.ds(keep_off, half), :]
                + recv_ref[pl.ds(_RS_OFFS[k], half), :]
            )
            base = keep_off

        for k in range(3):
            cur_len = CHUNK << k
            base_rows = ((my >> k) << k) * CHUNK
            partner = my ^ (1 << k)
            rdma = pltpu.make_async_remote_copy(
                src_ref=out_ref.at[pl.ds(base_rows, cur_len), :],
                dst_ref=out_ref.at[pl.ds(base_rows, cur_len), :],
                send_sem=send_sems.at[3 + k],
                recv_sem=recv_sems.at[3 + k],
                device_id=(partner,),
                device_id_type=pl.DeviceIdType.MESH,
            )
            rdma.start()
            rdma.wait()

    return pl.pallas_call(
        body,
        out_shape=jax.ShapeDtypeStruct((M, N), jnp.float32),
        in_specs=[
            pl.BlockSpec(memory_space=pltpu.VMEM),
            pl.BlockSpec(memory_space=pltpu.VMEM),
        ],
        out_specs=pl.BlockSpec(memory_space=pltpu.VMEM),
        scratch_shapes=[
            pltpu.VMEM((672, N), jnp.float32),
            pltpu.SemaphoreType.DMA((6,)),
            pltpu.SemaphoreType.DMA((6,)),
        ],
    )(A, B)


# device time: 33176 ns/iter; 2.0125x vs baseline; 2.0125x over previous
import jax
import jax.numpy as jnp
from jax import lax
from jax.experimental import pallas as pl
from jax.experimental.pallas import tpu as pltpu

N_DEV = 8
M = 768
N = 768

MASKS = (4, 3, 1)
G_ROWS = 256
G_RECV = 224
RS_RECV_OFF = (0, 128, 192)


def _vbit(my, m):
    if m == 4:
        return (my >> 2) & 1
    if m == 3:
        return (my >> 1) & 1
    return (my ^ (my >> 1)) & 1


def kernel(A, B):
    def body(a_ref, b_ref, out_ref, recv_ref, send_sems, recv_sems):
        my = lax.axis_index("i")

        out_ref[:, :] = jnp.dot(
            a_ref[:, :], b_ref[:, :], preferred_element_type=jnp.float32
        )

        barrier = pltpu.get_barrier_semaphore()
        for m in MASKS:
            pl.semaphore_signal(
                barrier, inc=1,
                device_id=(my ^ m,), device_id_type=pl.DeviceIdType.MESH,
            )
        pl.semaphore_wait(barrier, 3)

        bases = [jnp.int32(g * G_ROWS) for g in range(3)]

        for s in range(3):
            half = 128 >> s
            started = []
            for g in range(3):
                m = MASKS[(s + g) % 3]
                bit = _vbit(my, m)
                keep_off = bases[g] + bit * half
                send_off = bases[g] + (1 - bit) * half
                recv_off = g * G_RECV + RS_RECV_OFF[s]
                rdma = pltpu.make_async_remote_copy(
                    src_ref=out_ref.at[pl.ds(send_off, half), :],
                    dst_ref=recv_ref.at[pl.ds(recv_off, half), :],
                    send_sem=send_sems.at[g * 6 + s],
                    recv_sem=recv_sems.at[g * 6 + s],
                    device_id=(my ^ m,),
                    device_id_type=pl.DeviceIdType.MESH,
                )
                rdma.start()
                started.append((rdma, keep_off, recv_off))
                bases[g] = keep_off
            for rdma, keep_off, recv_off in started:
                rdma.wait()
                out_ref[pl.ds(keep_off, half), :] = (
                    out_ref[pl.ds(keep_off, half), :]
                    + recv_ref[pl.ds(recv_off, half), :]
                )

        for s in range(3):
            size = 32 << s
            started = []
            for g in range(3):
                m = MASKS[(2 - s + g) % 3]
                bit = _vbit(my, m)
                rdma = pltpu.make_async_remote_copy(
                    src_ref=out_ref.at[pl.ds(bases[g], size), :],
                    dst_ref=out_ref.at[pl.ds(bases[g], size), :],
                    send_sem=send_sems.at[g * 6 + 3 + s],
                    recv_sem=recv_sems.at[g * 6 + 3 + s],
                    device_id=(my ^ m,),
                    device_id_type=pl.DeviceIdType.MESH,
                )
                rdma.start()
                started.append(rdma)
                bases[g] = bases[g] - bit * size
            for rdma in started:
                rdma.wait()

    return pl.pallas_call(
        body,
        out_shape=jax.ShapeDtypeStruct((M, N), jnp.float32),
        in_specs=[
            pl.BlockSpec(memory_space=pltpu.VMEM),
            pl.BlockSpec(memory_space=pltpu.VMEM),
        ],
        out_specs=pl.BlockSpec(memory_space=pltpu.VMEM),
        scratch_shapes=[
            pltpu.VMEM((3 * G_RECV, N), jnp.float32),
            pltpu.SemaphoreType.DMA((18,)),
            pltpu.SemaphoreType.DMA((18,)),
        ],
        compiler_params=pltpu.CompilerParams(collective_id=0),
    )(A, B)
